# baseline (speedup 1.0000x reference)
import functools

import jax
import jax.numpy as jnp
from jax import lax
from jax.experimental import pallas as pl
from jax.experimental.pallas import tpu as pltpu
from jax.experimental.pallas import tpu_sc as plsc

_NC = 2
_NS = 16
_NW = _NC * _NS
_L = 16

_BATCH = 16384
_DIM = 128
_RPW = _BATCH // _NW
_CH = 128
_NCHUNKS = _RPW // _CH
_NSLICES = _DIM // _L


def _make_sc_film():
    mesh = plsc.VectorSubcoreMesh(core_axis_name="c", subcore_axis_name="s")

    @functools.partial(
        pl.kernel,
        mesh=mesh,
        out_type=jax.ShapeDtypeStruct((_BATCH, _DIM), jnp.float32),
        scratch_types=[
            pltpu.VMEM((2, _CH, _DIM), jnp.float32),
            pltpu.VMEM((2, _CH, _DIM), jnp.float32),
            pltpu.VMEM((2, _DIM), jnp.float32),
            pltpu.VMEM((2, _DIM), jnp.float32),
            pltpu.VMEM((_L,), jnp.float32),
            pltpu.SemaphoreType.DMA((2,)),
            pltpu.SemaphoreType.DMA((2,)),
        ],
    )
    def sc_film(x_hbm, g_hbm, b_hbm, t_hbm, o_hbm,
                xin, xout, gv, bv, tv, insem, outsem):
        wid = lax.axis_index("s") * _NC + lax.axis_index("c")
        base = wid * _RPW

        pltpu.sync_copy(g_hbm, gv)
        pltpu.sync_copy(b_hbm, bv)
        pltpu.sync_copy(t_hbm, tv)
        w1 = tv[...]
        gsel = []
        bsel = []
        for c in range(_NSLICES):
            g0 = gv[0, pl.ds(c * _L, _L)]
            g1 = gv[1, pl.ds(c * _L, _L)]
            b0 = bv[0, pl.ds(c * _L, _L)]
            b1 = bv[1, pl.ds(c * _L, _L)]
            gsel.append(g0 + (g1 - g0) * w1)
            bsel.append(b0 + (b1 - b0) * w1)

        def cin(k):
            return pltpu.make_async_copy(
                x_hbm.at[pl.ds(base + k * _CH, _CH), :],
                xin.at[k % 2], insem.at[k % 2])

        def cout(k):
            return pltpu.make_async_copy(
                xout.at[k % 2], o_hbm.at[pl.ds(base + k * _CH, _CH), :],
                outsem.at[k % 2])

        cin(0).start()
        for k in range(_NCHUNKS):
            if k + 1 < _NCHUNKS:
                cin(k + 1).start()
            cin(k).wait()
            if k >= 2:
                cout(k - 2).wait()
            s = k % 2

            def row_body(i, carry):
                for c in range(_NSLICES):
                    xv = xin[s, i, pl.ds(c * _L, _L)]
                    xout[s, i, pl.ds(c * _L, _L)] = xv * gsel[c] + bsel[c]
                return carry

            lax.fori_loop(0, _CH, row_body, 0)
            cout(k).start()
        cout(_NCHUNKS - 2).wait()
        cout(_NCHUNKS - 1).wait()

    return sc_film


_SC_FILM = _make_sc_film()


def kernel(x, gamma_w, beta_w, task_id):
    tt = jnp.full((_L,), jnp.asarray(task_id, jnp.float32))
    return _SC_FILM(x, gamma_w, beta_w, tt)


# speedup vs baseline: 3.1836x; 3.1836x over previous
import jax
import jax.numpy as jnp
from jax.experimental import pallas as pl
from jax.experimental.pallas import tpu as pltpu

_BLOCK_B = 4096


def _film_body(task_ref, x_ref, g_any, b_any, o_ref, gsel, bsel, sem):
    @pl.when(pl.program_id(0) == 0)
    def _():
        t = task_ref[0]
        pltpu.make_async_copy(g_any.at[pl.ds(t, 1), :], gsel, sem).start()
        pltpu.make_async_copy(g_any.at[pl.ds(t, 1), :], gsel, sem).wait()
        pltpu.make_async_copy(b_any.at[pl.ds(t, 1), :], bsel, sem).start()
        pltpu.make_async_copy(b_any.at[pl.ds(t, 1), :], bsel, sem).wait()

    o_ref[...] = x_ref[...] * gsel[...] + bsel[...]


def kernel(x, gamma_w, beta_w, task_id):
    batch, dim = x.shape
    task = jnp.asarray(task_id, dtype=jnp.int32).reshape((1,))
    block_b = min(_BLOCK_B, batch)
    grid = (batch // block_b,)
    return pl.pallas_call(
        _film_body,
        grid_spec=pltpu.PrefetchScalarGridSpec(
            num_scalar_prefetch=1,
            grid=grid,
            in_specs=[
                pl.BlockSpec((block_b, dim), lambda i, t: (i, 0)),
                pl.BlockSpec(memory_space=pl.ANY),
                pl.BlockSpec(memory_space=pl.ANY),
            ],
            out_specs=pl.BlockSpec((block_b, dim), lambda i, t: (i, 0)),
            scratch_shapes=[
                pltpu.VMEM((1, dim), jnp.float32),
                pltpu.VMEM((1, dim), jnp.float32),
                pltpu.SemaphoreType.DMA,
            ],
        ),
        out_shape=jax.ShapeDtypeStruct(x.shape, x.dtype),
        compiler_params=pltpu.CompilerParams(
            dimension_semantics=("arbitrary",),
        ),
    )(task, x, gamma_w, beta_w)


# speedup vs baseline: 4.4832x; 1.4082x over previous
import jax
import jax.numpy as jnp
from jax.experimental import pallas as pl
from jax.experimental.pallas import tpu as pltpu

_BLOCK_B = 8192


def _film_body(task_ref, x_ref, g_ref, b_ref, o_ref):
    del task_ref
    o_ref[...] = x_ref[...] * g_ref[0] + b_ref[0]


def kernel(x, gamma_w, beta_w, task_id):
    batch, dim = x.shape
    num_tasks = gamma_w.shape[0]
    task = jnp.asarray(task_id, dtype=jnp.int32).reshape((1,))
    g3 = gamma_w.reshape(num_tasks, 1, dim)
    b3 = beta_w.reshape(num_tasks, 1, dim)
    block_b = min(_BLOCK_B, batch)
    grid = (batch // block_b,)
    return pl.pallas_call(
        _film_body,
        grid_spec=pltpu.PrefetchScalarGridSpec(
            num_scalar_prefetch=1,
            grid=grid,
            in_specs=[
                pl.BlockSpec((block_b, dim), lambda i, t: (i, 0)),
                pl.BlockSpec((1, 1, dim), lambda i, t: (t[0], 0, 0)),
                pl.BlockSpec((1, 1, dim), lambda i, t: (t[0], 0, 0)),
            ],
            out_specs=pl.BlockSpec((block_b, dim), lambda i, t: (i, 0)),
        ),
        out_shape=jax.ShapeDtypeStruct(x.shape, x.dtype),
        compiler_params=pltpu.CompilerParams(
            dimension_semantics=("arbitrary",),
        ),
    )(task, x, g3, b3)


# speedup vs baseline: 4.5646x; 1.0182x over previous
import jax
import jax.numpy as jnp
from jax.experimental import pallas as pl
from jax.experimental.pallas import tpu as pltpu

_BLOCK_B = 8192


def _film_body(task_ref, x_ref, g_ref, b_ref, o_ref):
    del task_ref
    o_ref[...] = x_ref[...] * g_ref[0] + b_ref[0]


def kernel(x, gamma_w, beta_w, task_id):
    batch, dim = x.shape
    num_tasks = gamma_w.shape[0]
    task = jnp.asarray(task_id, dtype=jnp.int32).reshape((1,))
    g3 = gamma_w.reshape(num_tasks, 1, dim)
    b3 = beta_w.reshape(num_tasks, 1, dim)
    block_b = min(_BLOCK_B, batch)
    grid = (batch // block_b,)
    return pl.pallas_call(
        _film_body,
        grid_spec=pltpu.PrefetchScalarGridSpec(
            num_scalar_prefetch=1,
            grid=grid,
            in_specs=[
                pl.BlockSpec((block_b, dim), lambda i, t: (i, 0)),
                pl.BlockSpec((1, 1, dim), lambda i, t: (t[0], 0, 0)),
                pl.BlockSpec((1, 1, dim), lambda i, t: (t[0], 0, 0)),
            ],
            out_specs=pl.BlockSpec((block_b, dim), lambda i, t: (i, 0)),
        ),
        out_shape=jax.ShapeDtypeStruct(x.shape, x.dtype),
        compiler_params=pltpu.CompilerParams(
            dimension_semantics=("parallel",),
        ),
    )(task, x, g3, b3)
